# baseline (speedup 1.0000x reference)
import functools

import jax
import jax.numpy as jnp
from jax import lax
from jax.experimental import pallas as pl
from jax.experimental.pallas import tpu as pltpu
from jax.experimental.pallas import tpu_sc as plsc

_NUM_SEGMENTS = 32
_SEG_LENGTH = 16
_K = 3
_BN_EPS = 1e-5
_B = 64
_T = _NUM_SEGMENTS * _SEG_LENGTH
_D = 768
_C = 100
_CP = 128
_N = _B * _T

_VPB = 4
_ROWS = _VPB * _T
_NBLK = _N // _ROWS
_SPB = _VPB * _NUM_SEGMENTS


def _tc_body(lab_ref, text_ref, nc_ref, img_ref,
             norm_ref, raw_ref, stats_ref, keys_ref,
             rawv_ref, txt_ref, acc_ref, keyacc_ref):
    t = pl.program_id(0)
    i = pl.program_id(1)

    @pl.when((t == 0) & (i == 0))
    def _init():
        txt = text_ref[1:_C + 1, :] - nc_ref[0:1, :]
        nrm = jnp.sqrt(jnp.sum(txt * txt, axis=1, keepdims=True))
        txt_ref[0:_C, :] = txt / nrm
        txt_ref[_C:_CP, :] = jnp.zeros((_CP - _C, _D), jnp.float32)
        acc_ref[...] = jnp.zeros_like(acc_ref)

    @pl.when(t == 0)
    def _matmul():
        x = img_ref[...] - nc_ref[0:1, :]
        y = jax.lax.dot_general(x, txt_ref[...], (((1,), (1,)), ((), ())),
                                preferred_element_type=jnp.float32)
        rawv_ref[pl.ds(i * _ROWS, _ROWS), :] = y
        raw_ref[...] = y
        acc_ref[0:1, :] += jnp.sum(y, axis=0, keepdims=True)
        acc_ref[1:2, :] += jnp.sum(y * y, axis=0, keepdims=True)

        @pl.when(i == _NBLK - 1)
        def _fin():
            n = jnp.float32(_N)
            mean = acc_ref[0:1, :] / n
            var = acc_ref[1:2, :] / n - mean * mean
            acc_ref[2:3, :] = mean
            acc_ref[3:4, :] = 1.0 / jnp.sqrt(var + _BN_EPS)
            stats_ref[0:2, :] = acc_ref[2:4, :]
            stats_ref[2:8, :] = jnp.zeros((6, _CP), jnp.float32)

    @pl.when(t == 1)
    def _select():
        mean = acc_ref[2:3, :]
        inv = acc_ref[3:4, :]
        y = (rawv_ref[pl.ds(i * _ROWS, _ROWS), :] - mean) * inv
        norm_ref[...] = y[:, :_C]

        segsum = jnp.sum(y.reshape(_SPB, _SEG_LENGTH, _CP), axis=1)

        ridx = jax.lax.broadcasted_iota(jnp.int32, (_SPB, 1), 0)
        vid = ridx // _NUM_SEGMENTS
        labs = [lab_ref[i * _VPB + j] for j in range(_VPB)]
        alabs = [lab - (lab > 0).astype(lab.dtype) for lab in labs]
        lrow = alabs[_VPB - 1]
        for j in range(_VPB - 2, -1, -1):
            lrow = jnp.where(vid == j, alabs[j], lrow)
        lane = jax.lax.broadcasted_iota(jnp.int32, (_SPB, _CP), 1)
        onehot = (lane == lrow).astype(jnp.float32)
        w = jnp.where(i < (_B // 2) // _VPB, onehot, jnp.ones_like(onehot))
        key = jnp.sum(segsum * w, axis=1, keepdims=True)
        keyacc_ref[pl.ds(i * _VPB, _VPB), :] = key.reshape(_VPB, _NUM_SEGMENTS)

        @pl.when(i == _NBLK - 1)
        def _emit_keys():
            keys_ref[...] = keyacc_ref[...]


def _lane_reduce(x, largest):
    iota = lax.iota(jnp.int32, 16)
    for sh in (1, 2, 4, 8):
        xp = x.at[iota ^ sh].get(mode="promise_in_bounds")
        x = jnp.maximum(x, xp) if largest else jnp.minimum(x, xp)
    return x


def _sc_body(raw_hbm, keys_hbm, stats_hbm,
             topk_hbm, botk_hbm, tidx_hbm, bidx_hbm,
             kv, meanv, invv, buf, ivec, sem):
    wid = lax.axis_index("s") * 2 + lax.axis_index("c")
    pltpu.sync_copy(stats_hbm.at[0], meanv)
    pltpu.sync_copy(stats_hbm.at[1], invv)
    iota = lax.iota(jnp.int32, 16)

    def _one_video(vi):
        v = wid * 2 + vi
        pltpu.sync_copy(keys_hbm.at[v], kv)
        x0 = kv[pl.ds(0, 16)]
        x1 = kv[pl.ds(16, 16)]

        def _pick3(largest):
            c0, c1 = x0, x1
            fill = jnp.float32(-1e30) if largest else jnp.float32(1e30)
            picks = []
            for _ in range(_K):
                both = jnp.maximum(c0, c1) if largest else jnp.minimum(c0, c1)
                m = _lane_reduce(both, largest)
                cand0 = jnp.where(c0 == m, iota, _B)
                cand1 = jnp.where(c1 == m, iota + 16, _B)
                p = _lane_reduce(jnp.minimum(cand0, cand1), False)
                picks.append(p)
                c0 = jnp.where(iota == p, fill, c0)
                c1 = jnp.where(iota + 16 == p, fill, c1)
            return picks

        tis = _pick3(True)
        bis = _pick3(False)

        def _gather(picks, out_hbm):
            for k in range(_K):
                rows = v * _T + picks[k] * _SEG_LENGTH + iota
                pltpu.async_copy(raw_hbm.at[rows], buf, sem).wait()
                for r in range(_SEG_LENGTH):
                    def _chunk(c, _):
                        s = pl.ds(c * 16, 16)
                        buf[r, s] = (buf[r, s] - meanv[s]) * invv[s]
                        return 0
                    lax.fori_loop(0, _CP // 16, _chunk, 0)
                dst = (v * _K + k) * _SEG_LENGTH
                pltpu.sync_copy(buf, out_hbm.at[pl.ds(dst, _SEG_LENGTH)])

        _gather(tis, topk_hbm)
        _gather(bis, botk_hbm)

        def _emit_idx(picks, out_hbm):
            vec = jnp.where(iota == 0, picks[0],
                            jnp.where(iota == 1, picks[1],
                                      jnp.where(iota == 2, picks[2], 0)))
            ivec[...] = vec
            pltpu.sync_copy(ivec, out_hbm.at[v])

        _emit_idx(tis, tidx_hbm)
        _emit_idx(bis, bidx_hbm)

    _one_video(0)
    _one_video(1)


@functools.partial(jax.jit, static_argnames=("interpret",))
def _run(image_features, text_features, labels, ncentroid, interpret=False):
    img = image_features.reshape(_N, _D)
    nc = ncentroid.reshape(1, _D)

    grid_spec = pltpu.PrefetchScalarGridSpec(
        num_scalar_prefetch=1,
        grid=(2, _NBLK),
        in_specs=[
            pl.BlockSpec((_C + 1, _D), lambda t, i, lab: (0, 0)),
            pl.BlockSpec((1, _D), lambda t, i, lab: (0, 0)),
            pl.BlockSpec((_ROWS, _D),
                         lambda t, i, lab: (jnp.where(t == 0, i, _NBLK - 1), 0)),
        ],
        out_specs=[
            pl.BlockSpec((_ROWS, _C),
                         lambda t, i, lab: (jnp.where(t == 1, i, _NBLK - 1), 0)),
            pl.BlockSpec((_ROWS, _CP),
                         lambda t, i, lab: (jnp.where(t == 0, i, _NBLK - 1), 0)),
            pl.BlockSpec((8, _CP), lambda t, i, lab: (0, 0)),
            pl.BlockSpec((_B, _NUM_SEGMENTS), lambda t, i, lab: (0, 0)),
        ],
        scratch_shapes=[
            pltpu.VMEM((_N, _CP), jnp.float32),
            pltpu.VMEM((_CP, _D), jnp.float32),
            pltpu.VMEM((8, _CP), jnp.float32),
            pltpu.VMEM((_B, _NUM_SEGMENTS), jnp.float32),
        ],
    )
    norm, raw, stats, keys = pl.pallas_call(
        _tc_body,
        grid_spec=grid_spec,
        out_shape=[
            jax.ShapeDtypeStruct((_N, _C), jnp.float32),
            jax.ShapeDtypeStruct((_N, _CP), jnp.float32),
            jax.ShapeDtypeStruct((8, _CP), jnp.float32),
            jax.ShapeDtypeStruct((_B, _NUM_SEGMENTS), jnp.float32),
        ],
        interpret=interpret,
    )(labels.astype(jnp.int32), text_features, nc, img)

    mesh = plsc.VectorSubcoreMesh(core_axis_name="c", subcore_axis_name="s")
    sc = functools.partial(
        pl.kernel, mesh=mesh,
        out_type=[
            jax.ShapeDtypeStruct((_B * _K * _SEG_LENGTH, _CP), jnp.float32),
            jax.ShapeDtypeStruct((_B * _K * _SEG_LENGTH, _CP), jnp.float32),
            jax.ShapeDtypeStruct((_B, 16), jnp.int32),
            jax.ShapeDtypeStruct((_B, 16), jnp.int32),
        ],
        scratch_types=[
            pltpu.VMEM((_NUM_SEGMENTS,), jnp.float32),
            pltpu.VMEM((_CP,), jnp.float32),
            pltpu.VMEM((_CP,), jnp.float32),
            pltpu.VMEM((_SEG_LENGTH, _CP), jnp.float32),
            pltpu.VMEM((16,), jnp.int32),
            pltpu.SemaphoreType.DMA,
        ],
    )(_sc_body)
    topk, botk, tidx, bidx = sc(raw, keys, stats)

    return (norm, topk[:, :_C], botk[:, :_C],
            tidx[:_B // 2, :_K], tidx[_B // 2:, :_K], bidx[:_B // 2, :_K])


def kernel(image_features, text_features, labels, ncentroid, test_mode):
    return _run(image_features, text_features, labels, ncentroid)


# speedup vs baseline: 1.0227x; 1.0227x over previous
import functools

import jax
import jax.numpy as jnp
from jax import lax
from jax.experimental import pallas as pl
from jax.experimental.pallas import tpu as pltpu
from jax.experimental.pallas import tpu_sc as plsc

_NUM_SEGMENTS = 32
_SEG_LENGTH = 16
_K = 3
_BN_EPS = 1e-5
_B = 64
_T = _NUM_SEGMENTS * _SEG_LENGTH
_D = 768
_C = 100
_CP = 128
_N = _B * _T

_VPB = 4
_ROWS = _VPB * _T
_NBLK = _N // _ROWS
_SPB = _VPB * _NUM_SEGMENTS


def _tc_body(lab_ref, text_ref, nc_ref, img_ref,
             norm_ref, raw_ref, stats_ref, keys_ref,
             rawv_ref, txt_ref, acc_ref, keyacc_ref):
    t = pl.program_id(0)
    i = pl.program_id(1)

    @pl.when((t == 0) & (i == 0))
    def _init():
        txt = text_ref[1:_C + 1, :] - nc_ref[0:1, :]
        nrm = jnp.sqrt(jnp.sum(txt * txt, axis=1, keepdims=True))
        txt_ref[0:_C, :] = txt / nrm
        txt_ref[_C:_CP, :] = jnp.zeros((_CP - _C, _D), jnp.float32)
        acc_ref[...] = jnp.zeros_like(acc_ref)

    @pl.when(t == 0)
    def _matmul():
        x = img_ref[...] - nc_ref[0:1, :]
        y = jax.lax.dot_general(x, txt_ref[...], (((1,), (1,)), ((), ())),
                                preferred_element_type=jnp.float32)
        rawv_ref[pl.ds(i * _ROWS, _ROWS), :] = y
        raw_ref[...] = y
        acc_ref[0:1, :] += jnp.sum(y, axis=0, keepdims=True)
        acc_ref[1:2, :] += jnp.sum(y * y, axis=0, keepdims=True)

        @pl.when(i == _NBLK - 1)
        def _fin():
            n = jnp.float32(_N)
            mean = acc_ref[0:1, :] / n
            var = acc_ref[1:2, :] / n - mean * mean
            acc_ref[2:3, :] = mean
            acc_ref[3:4, :] = 1.0 / jnp.sqrt(var + _BN_EPS)
            stats_ref[0:2, :] = acc_ref[2:4, :]
            stats_ref[2:8, :] = jnp.zeros((6, _CP), jnp.float32)

    @pl.when(t == 1)
    def _select():
        mean = acc_ref[2:3, :]
        inv = acc_ref[3:4, :]
        y = (rawv_ref[pl.ds(i * _ROWS, _ROWS), :] - mean) * inv
        norm_ref[...] = y[:, :_C]

        segsum = jnp.sum(y.reshape(_SPB, _SEG_LENGTH, _CP), axis=1)

        ridx = jax.lax.broadcasted_iota(jnp.int32, (_SPB, 1), 0)
        vid = ridx // _NUM_SEGMENTS
        labs = [lab_ref[i * _VPB + j] for j in range(_VPB)]
        alabs = [lab - (lab > 0).astype(lab.dtype) for lab in labs]
        lrow = alabs[_VPB - 1]
        for j in range(_VPB - 2, -1, -1):
            lrow = jnp.where(vid == j, alabs[j], lrow)
        lane = jax.lax.broadcasted_iota(jnp.int32, (_SPB, _CP), 1)
        onehot = (lane == lrow).astype(jnp.float32)
        w = jnp.where(i < (_B // 2) // _VPB, onehot, jnp.ones_like(onehot))
        key = jnp.sum(segsum * w, axis=1, keepdims=True)
        keyacc_ref[pl.ds(i * _VPB, _VPB), :] = key.reshape(_VPB, _NUM_SEGMENTS)

        @pl.when(i == _NBLK - 1)
        def _emit_keys():
            keys_ref[...] = keyacc_ref[...]


def _lane_reduce(x, largest):
    iota = lax.iota(jnp.int32, 16)
    for sh in (1, 2, 4, 8):
        xp = x.at[iota ^ sh].get(mode="promise_in_bounds")
        x = jnp.maximum(x, xp) if largest else jnp.minimum(x, xp)
    return x


def _sc_body(raw_hbm, keys_hbm, stats_hbm,
             topk_hbm, botk_hbm, tidx_hbm, bidx_hbm,
             kv, meanv, invv, buf, ivec, sem):
    wid = lax.axis_index("s") * 2 + lax.axis_index("c")
    pltpu.sync_copy(stats_hbm.at[0], meanv)
    pltpu.sync_copy(stats_hbm.at[1], invv)
    iota = lax.iota(jnp.int32, 16)

    def _one_video(vi):
        v = wid * 2 + vi
        pltpu.sync_copy(keys_hbm.at[v], kv)
        x0 = kv[pl.ds(0, 16)]
        x1 = kv[pl.ds(16, 16)]

        def _pick3(largest):
            c0, c1 = x0, x1
            fill = jnp.float32(-1e30) if largest else jnp.float32(1e30)
            picks = []
            for _ in range(_K):
                both = jnp.maximum(c0, c1) if largest else jnp.minimum(c0, c1)
                m = _lane_reduce(both, largest)
                cand0 = jnp.where(c0 == m, iota, _B)
                cand1 = jnp.where(c1 == m, iota + 16, _B)
                p = _lane_reduce(jnp.minimum(cand0, cand1), False)
                picks.append(p)
                c0 = jnp.where(iota == p, fill, c0)
                c1 = jnp.where(iota + 16 == p, fill, c1)
            return picks

        tis = _pick3(True)
        bis = _pick3(False)

        ins = []
        for k in range(_K):
            for slot, picks in ((k, tis), (k + _K, bis)):
                rows = v * _T + picks[k] * _SEG_LENGTH + iota
                dstv = buf.at[pl.ds(slot * _SEG_LENGTH, _SEG_LENGTH)]
                ins.append(pltpu.async_copy(raw_hbm.at[rows], dstv, sem))
        for cp in ins:
            cp.wait()
        for r in range(2 * _K * _SEG_LENGTH):
            def _chunk(c, _):
                s = pl.ds(c * 16, 16)
                buf[r, s] = (buf[r, s] - meanv[s]) * invv[s]
                return 0
            lax.fori_loop(0, _CP // 16, _chunk, 0)
        outs = []
        for k in range(_K):
            for slot, out_hbm in ((k, topk_hbm), (k + _K, botk_hbm)):
                dst = (v * _K + k) * _SEG_LENGTH
                srcv = buf.at[pl.ds(slot * _SEG_LENGTH, _SEG_LENGTH)]
                outs.append(pltpu.async_copy(srcv, out_hbm.at[pl.ds(dst, _SEG_LENGTH)], sem))
        for cp in outs:
            cp.wait()

        def _emit_idx(picks, out_hbm):
            vec = jnp.where(iota == 0, picks[0],
                            jnp.where(iota == 1, picks[1],
                                      jnp.where(iota == 2, picks[2], 0)))
            ivec[...] = vec
            pltpu.sync_copy(ivec, out_hbm.at[v])

        _emit_idx(tis, tidx_hbm)
        _emit_idx(bis, bidx_hbm)

    _one_video(0)
    _one_video(1)


@functools.partial(jax.jit, static_argnames=("interpret",))
def _run(image_features, text_features, labels, ncentroid, interpret=False):
    img = image_features.reshape(_N, _D)
    nc = ncentroid.reshape(1, _D)

    grid_spec = pltpu.PrefetchScalarGridSpec(
        num_scalar_prefetch=1,
        grid=(2, _NBLK),
        in_specs=[
            pl.BlockSpec((_C + 1, _D), lambda t, i, lab: (0, 0)),
            pl.BlockSpec((1, _D), lambda t, i, lab: (0, 0)),
            pl.BlockSpec((_ROWS, _D),
                         lambda t, i, lab: (jnp.where(t == 0, i, _NBLK - 1), 0)),
        ],
        out_specs=[
            pl.BlockSpec((_ROWS, _C),
                         lambda t, i, lab: (jnp.where(t == 1, i, _NBLK - 1), 0)),
            pl.BlockSpec((_ROWS, _CP),
                         lambda t, i, lab: (jnp.where(t == 0, i, _NBLK - 1), 0)),
            pl.BlockSpec((8, _CP), lambda t, i, lab: (0, 0)),
            pl.BlockSpec((_B, _NUM_SEGMENTS), lambda t, i, lab: (0, 0)),
        ],
        scratch_shapes=[
            pltpu.VMEM((_N, _CP), jnp.float32),
            pltpu.VMEM((_CP, _D), jnp.float32),
            pltpu.VMEM((8, _CP), jnp.float32),
            pltpu.VMEM((_B, _NUM_SEGMENTS), jnp.float32),
        ],
    )
    norm, raw, stats, keys = pl.pallas_call(
        _tc_body,
        grid_spec=grid_spec,
        out_shape=[
            jax.ShapeDtypeStruct((_N, _C), jnp.float32),
            jax.ShapeDtypeStruct((_N, _CP), jnp.float32),
            jax.ShapeDtypeStruct((8, _CP), jnp.float32),
            jax.ShapeDtypeStruct((_B, _NUM_SEGMENTS), jnp.float32),
        ],
        interpret=interpret,
    )(labels.astype(jnp.int32), text_features, nc, img)

    mesh = plsc.VectorSubcoreMesh(core_axis_name="c", subcore_axis_name="s")
    sc = functools.partial(
        pl.kernel, mesh=mesh,
        out_type=[
            jax.ShapeDtypeStruct((_B * _K * _SEG_LENGTH, _CP), jnp.float32),
            jax.ShapeDtypeStruct((_B * _K * _SEG_LENGTH, _CP), jnp.float32),
            jax.ShapeDtypeStruct((_B, 16), jnp.int32),
            jax.ShapeDtypeStruct((_B, 16), jnp.int32),
        ],
        scratch_types=[
            pltpu.VMEM((_NUM_SEGMENTS,), jnp.float32),
            pltpu.VMEM((_CP,), jnp.float32),
            pltpu.VMEM((_CP,), jnp.float32),
            pltpu.VMEM((2 * _K * _SEG_LENGTH, _CP), jnp.float32),
            pltpu.VMEM((16,), jnp.int32),
            pltpu.SemaphoreType.DMA,
        ],
    )(_sc_body)
    topk, botk, tidx, bidx = sc(raw, keys, stats)

    return (norm, topk[:, :_C], botk[:, :_C],
            tidx[:_B // 2, :_K], tidx[_B // 2:, :_K], bidx[:_B // 2, :_K])


def kernel(image_features, text_features, labels, ncentroid, test_mode):
    return _run(image_features, text_features, labels, ncentroid)
